# baseline (speedup 1.0000x reference)
import functools

import jax
import jax.numpy as jnp
from jax.experimental import pallas as pl
from jax.experimental.pallas import tpu as pltpu

_LANES = 128
_SUBLANES = 8
_CHUNK_ROWS = 2048


def _rmse_kernel(yhat_ref, y_ref, out_ref, *, n_chunks, inv_n, eps):
    acc = jnp.zeros((_SUBLANES, _LANES), jnp.float32)
    for c in range(n_chunks):
        ych = yhat_ref[pl.ds(c * _CHUNK_ROWS, _CHUNK_ROWS), :]
        tch = y_ref[pl.ds(c * _CHUNK_ROWS, _CHUNK_ROWS), :]
        d = ych - tch
        sq = d * d
        acc = acc + jnp.sum(sq.reshape(-1, _SUBLANES, _LANES), axis=0)

    total = jnp.sum(acc)
    out_ref[0, 0] = jnp.sqrt(total * jnp.float32(inv_n) + jnp.float32(eps))


@functools.partial(jax.jit, static_argnames=("eps",))
def _rmse(yhat, y, eps=1e-6):
    n_elems = yhat.size
    flat_yhat = yhat.reshape(-1)
    flat_y = y.reshape(-1)

    chunk = _CHUNK_ROWS * _LANES
    n_pad = (-n_elems) % chunk
    if n_pad:
        flat_yhat = jnp.pad(flat_yhat, (0, n_pad))
        flat_y = jnp.pad(flat_y, (0, n_pad))

    n_rows = (n_elems + n_pad) // _LANES
    n_chunks = n_rows // _CHUNK_ROWS

    yhat2d = flat_yhat.reshape(n_rows, _LANES).astype(jnp.float32)
    y2d = flat_y.reshape(n_rows, _LANES).astype(jnp.float32)

    body = functools.partial(
        _rmse_kernel, n_chunks=n_chunks, inv_n=1.0 / n_elems, eps=float(eps))

    yhat_v = pltpu.with_memory_space_constraint(yhat2d, pltpu.VMEM)
    y_v = pltpu.with_memory_space_constraint(y2d, pltpu.VMEM)

    out = pl.pallas_call(
        body,
        out_shape=jax.ShapeDtypeStruct((1, 1), jnp.float32),
        in_specs=[
            pl.BlockSpec(memory_space=pltpu.VMEM),
            pl.BlockSpec(memory_space=pltpu.VMEM),
        ],
        out_specs=pl.BlockSpec(memory_space=pltpu.SMEM),
        compiler_params=pltpu.CompilerParams(
            vmem_limit_bytes=16 * 1024 * 1024),
    )(yhat_v, y_v)

    return out.reshape(())


def kernel(yhat, y):
    return _rmse(yhat, y)


# speedup vs baseline: 1.0408x; 1.0408x over previous
import functools

import jax
import jax.numpy as jnp
from jax.experimental import pallas as pl
from jax.experimental.pallas import tpu as pltpu

_LANES = 128
_SUBLANES = 8
_CHUNK_ROWS = 8192
_NUM_BUFS = 3


def _rmse_kernel(yhat_hbm, y_hbm, out_ref, ybuf, tbuf, ysem, tsem,
                 *, n_chunks, inv_n, eps):
    def copy_in(c):
        s = c % _NUM_BUFS
        r0 = c * _CHUNK_ROWS
        return (
            pltpu.make_async_copy(
                yhat_hbm.at[pl.ds(r0, _CHUNK_ROWS), :], ybuf.at[s],
                ysem.at[s]),
            pltpu.make_async_copy(
                y_hbm.at[pl.ds(r0, _CHUNK_ROWS), :], tbuf.at[s],
                tsem.at[s]),
        )

    for c in range(min(_NUM_BUFS, n_chunks)):
        a, b = copy_in(c)
        a.start()
        b.start()

    acc = jnp.zeros((_SUBLANES, _LANES), jnp.float32)
    for c in range(n_chunks):
        s = c % _NUM_BUFS
        a, b = copy_in(c)
        a.wait()
        b.wait()
        d = ybuf[s] - tbuf[s]
        sq = d * d
        acc = acc + jnp.sum(sq.reshape(-1, _SUBLANES, _LANES), axis=0)
        if c + _NUM_BUFS < n_chunks:
            a, b = copy_in(c + _NUM_BUFS)
            a.start()
            b.start()

    total = jnp.sum(acc)
    out_ref[0, 0] = jnp.sqrt(total * jnp.float32(inv_n) + jnp.float32(eps))


@functools.partial(jax.jit, static_argnames=("eps",))
def _rmse(yhat, y, eps=1e-6):
    n_elems = yhat.size
    flat_yhat = yhat.reshape(-1)
    flat_y = y.reshape(-1)

    chunk = _CHUNK_ROWS * _LANES
    n_pad = (-n_elems) % chunk
    if n_pad:
        flat_yhat = jnp.pad(flat_yhat, (0, n_pad))
        flat_y = jnp.pad(flat_y, (0, n_pad))

    n_rows = (n_elems + n_pad) // _LANES
    n_chunks = n_rows // _CHUNK_ROWS

    yhat2d = flat_yhat.reshape(n_rows, _LANES).astype(jnp.float32)
    y2d = flat_y.reshape(n_rows, _LANES).astype(jnp.float32)

    body = functools.partial(
        _rmse_kernel, n_chunks=n_chunks, inv_n=1.0 / n_elems, eps=float(eps))

    out = pl.pallas_call(
        body,
        out_shape=jax.ShapeDtypeStruct((1, 1), jnp.float32),
        in_specs=[
            pl.BlockSpec(memory_space=pl.ANY),
            pl.BlockSpec(memory_space=pl.ANY),
        ],
        out_specs=pl.BlockSpec(memory_space=pltpu.SMEM),
        scratch_shapes=[
            pltpu.VMEM((_NUM_BUFS, _CHUNK_ROWS, _LANES), jnp.float32),
            pltpu.VMEM((_NUM_BUFS, _CHUNK_ROWS, _LANES), jnp.float32),
            pltpu.SemaphoreType.DMA((_NUM_BUFS,)),
            pltpu.SemaphoreType.DMA((_NUM_BUFS,)),
        ],
    )(yhat2d, y2d)

    return out.reshape(())


def kernel(yhat, y):
    return _rmse(yhat, y)


# speedup vs baseline: 1.0506x; 1.0094x over previous
import functools

import jax
import jax.numpy as jnp
from jax.experimental import pallas as pl
from jax.experimental.pallas import tpu as pltpu

_LANES = 128
_SUBLANES = 8
_CHUNK_ROWS = 2048
_NUM_BUFS = 8


def _rmse_kernel(yhat_hbm, y_hbm, out_ref, ybuf, tbuf, ysem, tsem,
                 *, n_chunks, inv_n, eps):
    def copy_in(c):
        s = c % _NUM_BUFS
        r0 = c * _CHUNK_ROWS
        return (
            pltpu.make_async_copy(
                yhat_hbm.at[pl.ds(r0, _CHUNK_ROWS), :], ybuf.at[s],
                ysem.at[s]),
            pltpu.make_async_copy(
                y_hbm.at[pl.ds(r0, _CHUNK_ROWS), :], tbuf.at[s],
                tsem.at[s]),
        )

    for c in range(min(_NUM_BUFS, n_chunks)):
        a, b = copy_in(c)
        a.start()
        b.start()

    acc = jnp.zeros((_SUBLANES, _LANES), jnp.float32)
    for c in range(n_chunks):
        s = c % _NUM_BUFS
        a, b = copy_in(c)
        a.wait()
        b.wait()
        d = ybuf[s] - tbuf[s]
        sq = d * d
        acc = acc + jnp.sum(sq.reshape(-1, _SUBLANES, _LANES), axis=0)
        if c + _NUM_BUFS < n_chunks:
            a, b = copy_in(c + _NUM_BUFS)
            a.start()
            b.start()

    total = jnp.sum(acc)
    out_ref[0, 0] = jnp.sqrt(total * jnp.float32(inv_n) + jnp.float32(eps))


@functools.partial(jax.jit, static_argnames=("eps",))
def _rmse(yhat, y, eps=1e-6):
    n_elems = yhat.size
    flat_yhat = yhat.reshape(-1)
    flat_y = y.reshape(-1)

    chunk = _CHUNK_ROWS * _LANES
    n_pad = (-n_elems) % chunk
    if n_pad:
        flat_yhat = jnp.pad(flat_yhat, (0, n_pad))
        flat_y = jnp.pad(flat_y, (0, n_pad))

    n_rows = (n_elems + n_pad) // _LANES
    n_chunks = n_rows // _CHUNK_ROWS

    yhat2d = flat_yhat.reshape(n_rows, _LANES).astype(jnp.float32)
    y2d = flat_y.reshape(n_rows, _LANES).astype(jnp.float32)

    body = functools.partial(
        _rmse_kernel, n_chunks=n_chunks, inv_n=1.0 / n_elems, eps=float(eps))

    out = pl.pallas_call(
        body,
        out_shape=jax.ShapeDtypeStruct((1, 1), jnp.float32),
        in_specs=[
            pl.BlockSpec(memory_space=pl.ANY),
            pl.BlockSpec(memory_space=pl.ANY),
        ],
        out_specs=pl.BlockSpec(memory_space=pltpu.SMEM),
        scratch_shapes=[
            pltpu.VMEM((_NUM_BUFS, _CHUNK_ROWS, _LANES), jnp.float32),
            pltpu.VMEM((_NUM_BUFS, _CHUNK_ROWS, _LANES), jnp.float32),
            pltpu.SemaphoreType.DMA((_NUM_BUFS,)),
            pltpu.SemaphoreType.DMA((_NUM_BUFS,)),
        ],
    )(yhat2d, y2d)

    return out.reshape(())


def kernel(yhat, y):
    return _rmse(yhat, y)
